# Initial kernel scaffold; baseline (speedup 1.0000x reference)
import jax
import jax.numpy as jnp
from jax.experimental import pallas as pl


def kernel(user_hidden_out, item_hidden_out, fake_user_hidden_out, fake_item_hidden_out, UV_adj, VU_adj, CUV_adj, CVU_adj, user_One, item_One, Wgu, bgu, Wgv, bgv, W_lin, b_lin, W_sub, b_sub, W_disc, b_disc):
    raise NotImplementedError("write your pallas kernel here")


# speedup vs baseline: 3.0836x; 3.0836x over previous
import jax
import jax.numpy as jnp
from jax import lax
from jax.experimental import pallas as pl
from jax.experimental.pallas import tpu as pltpu
from jax.experimental.pallas import tpu_sc as plsc

N = 10000
E = 320000
D = 128
B = 4096
NPAD = N + 240
NCROW = 80
NC = 2
NS = 16
CH = 128
NCHUNK = 157
EPAD = NS * NCHUNK * CH
NW = NC * NS
BPW = B // NW


def _sc_mesh():
    return plsc.VectorSubcoreMesh(
        core_axis_name="c", subcore_axis_name="s", num_cores=NC, num_subcores=NS
    )


def _segsum_body(item_f, fitem_f, user_f, fuser_f,
                 d0, s0, d1, s1, d2, s2, d3, s3, zinit, eye_hbm,
                 a0, a1, a2, a3, c0, c1, c2, c3,
                 acc, cnt, eye_s, didxb, sidxb, rows, oneh,
                 cidx, ridx, sem):
    c = lax.axis_index("c")
    s = lax.axis_index("s")
    rows_per_tile = NPAD // NS
    base = s * rows_per_tile

    @pl.when(s == 0)
    def _():
        pltpu.sync_copy(eye_hbm, eye_s)

    def run_task(dst_hbm, src_hbm, feat_hbm, out_hbm, cnt_hbm):
        pltpu.sync_copy(zinit, rows)

        def zero_body(i, _):
            off = pl.multiple_of(base + i * 128, 128)
            pltpu.sync_copy(rows, acc.at[pl.ds(off, 128)])
            return ()

        lax.fori_loop(0, rows_per_tile // 128, zero_body, (), unroll=False)

        @pl.when(s == 0)
        def _():
            pltpu.sync_copy(rows.at[pl.ds(0, NCROW)], cnt)

        plsc.subcore_barrier()

        def body(j, _):
            pltpu.sync_copy(dst_hbm.at[s, j], didxb)
            pltpu.sync_copy(src_hbm.at[s, j], sidxb)
            pltpu.async_copy(feat_hbm.at[sidxb], rows, sem).wait()
            pltpu.sync_copy(rows, acc.at[didxb], add=True)
            for k in range(CH // 16):
                d16 = didxb[pl.ds(k * 16, 16)]
                cidx[pl.ds(k * 16, 16)] = jnp.bitwise_and(d16, 127)
                ridx[pl.ds(k * 16, 16)] = jnp.right_shift(d16, 7)
            pltpu.async_copy(eye_s.at[cidx], oneh, sem).wait()
            pltpu.sync_copy(oneh, cnt.at[ridx], add=True)
            return ()

        lax.fori_loop(0, NCHUNK, body, (), unroll=False)
        plsc.subcore_barrier()

        def flush_body(i, _):
            off = pl.multiple_of(base + i * 128, 128)
            pltpu.sync_copy(acc.at[pl.ds(off, 128)],
                            out_hbm.at[pl.ds(off, 128)])
            return ()

        lax.fori_loop(0, rows_per_tile // 128, flush_body, (), unroll=False)

        @pl.when(s < 5)
        def _():
            csl = pl.ds(s * 16, 16)
            pltpu.sync_copy(cnt.at[csl], cnt_hbm.at[csl])

    @pl.when(c == 0)
    def _():
        run_task(d0, s0, item_f, a0, c0)
        run_task(d1, s1, fitem_f, a1, c1)

    @pl.when(c == 1)
    def _():
        run_task(d2, s2, user_f, a2, c2)
        run_task(d3, s3, fuser_f, a3, c3)


def _segsum_call(item_f, fitem_f, user_f, fuser_f,
                 d0, s0, d1, s1, d2, s2, d3, s3, zinit, eye):
    outa = jax.ShapeDtypeStruct((NPAD, D), jnp.float32)
    outc = jax.ShapeDtypeStruct((NCROW, 128), jnp.float32)
    f = pl.kernel(
        _segsum_body,
        out_type=(outa, outa, outa, outa, outc, outc, outc, outc),
        mesh=_sc_mesh(),
        scratch_types=[
            pltpu.VMEM_SHARED((NPAD, D), jnp.float32),
            pltpu.VMEM_SHARED((NCROW, 128), jnp.float32),
            pltpu.VMEM_SHARED((CH, 128), jnp.float32),
            pltpu.VMEM((CH,), jnp.int32),
            pltpu.VMEM((CH,), jnp.int32),
            pltpu.VMEM((CH, D), jnp.float32),
            pltpu.VMEM((CH, 128), jnp.float32),
            pltpu.VMEM((CH,), jnp.int32),
            pltpu.VMEM((CH,), jnp.int32),
            pltpu.SemaphoreType.DMA,
        ],
    )
    return f(item_f, fitem_f, user_f, fuser_f,
             d0, s0, d1, s1, d2, s2, d3, s3, zinit, eye)


def _gather_body(a0, a1, a2, a3, cn0, cn1, cn2, cn3, uidx_hbm, iidx_hbm,
                 g0, g1, g2, g3, o0, o1, o2, o3,
                 idxu, idxi, rows, cntv, cbuf, sem):
    c = lax.axis_index("c")
    s = lax.axis_index("s")
    w = c * NS + s
    pltpu.sync_copy(uidx_hbm.at[w], idxu)
    pltpu.sync_copy(iidx_hbm.at[w], idxi)

    def one_table(acc_hbm, cnt_hbm, idx_ref, out_hbm, co_hbm):
        pltpu.async_copy(acc_hbm.at[idx_ref], rows, sem).wait()
        pltpu.sync_copy(rows, out_hbm.at[pl.ds(w * BPW, BPW)])
        pltpu.sync_copy(cnt_hbm, cntv)
        for g in range(BPW // 16):
            iv = idx_ref[pl.ds(g * 16, 16)]
            cv = plsc.load_gather(cntv, [iv])
            cbuf[pl.ds(g * 16, 16)] = cv
        pltpu.sync_copy(cbuf, co_hbm.at[w])

    one_table(a0, cn0, idxu, g0, o0)
    one_table(a1, cn1, idxu, g1, o1)
    one_table(a2, cn2, idxi, g2, o2)
    one_table(a3, cn3, idxi, g3, o3)


def _gather_call(a0, a1, a2, a3, cn0, cn1, cn2, cn3, uidx, iidx):
    outg = jax.ShapeDtypeStruct((B, D), jnp.float32)
    outc = jax.ShapeDtypeStruct((NW, BPW), jnp.float32)
    f = pl.kernel(
        _gather_body,
        out_type=(outg, outg, outg, outg, outc, outc, outc, outc),
        mesh=_sc_mesh(),
        compiler_params=pltpu.CompilerParams(needs_layout_passes=False),
        scratch_types=[
            pltpu.VMEM((BPW,), jnp.int32),
            pltpu.VMEM((BPW,), jnp.int32),
            pltpu.VMEM((BPW, D), jnp.float32),
            pltpu.VMEM((NPAD,), jnp.float32),
            pltpu.VMEM((BPW,), jnp.float32),
            pltpu.SemaphoreType.DMA,
        ],
    )
    return f(a0, a1, a2, a3, cn0, cn1, cn2, cn3, uidx, iidx)


def _dense_body(u_ref, i_ref, g0, g1, g2, g3, c0, c1, c2, c3,
                wgut, bgu, wgvt, bgv, wlint, blin, wsubt, bsub, wdt, bd,
                prob_ref, label_ref):
    f32 = jnp.float32
    su = jnp.mean(u_ref[...], axis=0, keepdims=True)
    si = jnp.mean(i_ref[...], axis=0, keepdims=True)
    scat = jnp.concatenate([su, si], axis=1)
    s_two = jax.nn.sigmoid(
        jnp.dot(scat, wlint[...], preferred_element_type=f32) + blin[...])
    t = jnp.dot(s_two, wdt[...], preferred_element_type=f32)

    def gat(g, cn, wt, b):
        m = g[...] / jnp.maximum(cn[...], 1.0)
        return jax.nn.relu(jnp.dot(m, wt[...], preferred_element_type=f32) + b[...])

    ru = gat(g0, c0, wgut, bgu)
    fu = gat(g1, c1, wgut, bgu)
    ri = gat(g2, c2, wgvt, bgv)
    fi = gat(g3, c3, wgvt, bgv)

    wsu = wsubt[0:D, :]
    wsi = wsubt[D:2 * D, :]
    real_sub = jax.nn.sigmoid(
        jnp.dot(ru, wsu, preferred_element_type=f32)
        + jnp.dot(ri, wsi, preferred_element_type=f32) + bsub[...])
    fake_sub = jax.nn.sigmoid(
        jnp.dot(fu, wsu, preferred_element_type=f32)
        + jnp.dot(fi, wsi, preferred_element_type=f32) + bsub[...])

    b0 = bd[0, 0]
    real_prob = jax.nn.sigmoid(
        jnp.sum(real_sub * t, axis=1, keepdims=True) + b0)
    fake_prob = jax.nn.sigmoid(
        jnp.sum(fake_sub * t, axis=1, keepdims=True) + b0)
    prob_ref[pl.ds(0, B), :] = real_prob
    prob_ref[pl.ds(B, B), :] = fake_prob
    label_ref[pl.ds(0, B), :] = jnp.ones((B, 1), f32)
    label_ref[pl.ds(B, B), :] = jnp.zeros((B, 1), f32)


def _dense_call(uh, ih, g0, g1, g2, g3, c0, c1, c2, c3,
                wgut, bgu, wgvt, bgv, wlint, blin, wsubt, bsub, wdt, bd):
    return pl.pallas_call(
        _dense_body,
        out_shape=(
            jax.ShapeDtypeStruct((2 * B, 1), jnp.float32),
            jax.ShapeDtypeStruct((2 * B, 1), jnp.float32),
        ),
    )(uh, ih, g0, g1, g2, g3, c0, c1, c2, c3,
      wgut, bgu, wgvt, bgv, wlint, blin, wsubt, bsub, wdt, bd)


def kernel(user_hidden_out, item_hidden_out, fake_user_hidden_out,
           fake_item_hidden_out, UV_adj, VU_adj, CUV_adj, CVU_adj,
           user_One, item_One, Wgu, bgu, Wgv, bgv, W_lin, b_lin,
           W_sub, b_sub, W_disc, b_disc):
    f32 = jnp.float32

    def prep(adj):
        pad = EPAD - E
        dst = jnp.concatenate(
            [adj[0].astype(jnp.int32), jnp.full((pad,), N, jnp.int32)])
        src = jnp.concatenate(
            [adj[1].astype(jnp.int32), jnp.zeros((pad,), jnp.int32)])
        return dst.reshape(NS, NCHUNK, CH), src.reshape(NS, NCHUNK, CH)

    d0, s0 = prep(UV_adj)
    d1, s1 = prep(CUV_adj)
    d2, s2 = prep(VU_adj)
    d3, s3 = prep(CVU_adj)
    zinit = jnp.zeros((128, 128), f32)
    eye = jnp.eye(CH, dtype=f32)

    a0, a1, a2, a3, c0, c1, c2, c3 = _segsum_call(
        item_hidden_out, fake_item_hidden_out,
        user_hidden_out, fake_user_hidden_out,
        d0, s0, d1, s1, d2, s2, d3, s3, zinit, eye)

    uidx = user_One.astype(jnp.int32).reshape(NW, BPW)
    iidx = item_One.astype(jnp.int32).reshape(NW, BPW)
    g0, g1, g2, g3, o0, o1, o2, o3 = _gather_call(
        a0, a1, a2, a3,
        c0.reshape(NPAD), c1.reshape(NPAD), c2.reshape(NPAD), c3.reshape(NPAD),
        uidx, iidx)

    prob2, label2 = _dense_call(
        user_hidden_out, item_hidden_out, g0, g1, g2, g3,
        o0.reshape(B, 1), o1.reshape(B, 1), o2.reshape(B, 1), o3.reshape(B, 1),
        Wgu.T, bgu.reshape(1, D), Wgv.T, bgv.reshape(1, D),
        W_lin.T, b_lin.reshape(1, D), W_sub.T, b_sub.reshape(1, D),
        W_disc[0].T, b_disc.reshape(1, 1))

    return prob2.reshape(2 * B), label2.reshape(2 * B)


# speedup vs baseline: 6.3697x; 2.0656x over previous
import jax
import jax.numpy as jnp
from jax import lax
from jax.experimental import pallas as pl
from jax.experimental.pallas import tpu as pltpu
from jax.experimental.pallas import tpu_sc as plsc

N = 10000
E = 320000
D = 128
B = 4096
NPAD = N + 112
NCROW = 80
NC = 2
NS = 16
CH = 128
NCHUNK = 157
EPAD = NS * NCHUNK * CH
NW = NC * NS
BPW = B // NW


def _sc_mesh():
    return plsc.VectorSubcoreMesh(
        core_axis_name="c", subcore_axis_name="s", num_cores=NC, num_subcores=NS
    )


def _segsum_body(item_f, fitem_f, user_f, fuser_f,
                 d0, s0, d1, s1, d2, s2, d3, s3, zinit, zeros1d,
                 a0, a1, a2, a3, c0, c1, c2, c3,
                 acc, cnt_stage,
                 dA, dB, sA, sB, rA, rB,
                 cnt1d, rtmp, racc,
                 gsemA, gsemB, ssemA, ssemB):
    c = lax.axis_index("c")
    s = lax.axis_index("s")
    rows_per_tile = NPAD // NS
    base = s * rows_per_tile
    bufs = [(dA, sA, rA, gsemA, ssemA), (dB, sB, rB, gsemB, ssemB)]
    nbuf = len(bufs)

    def run_task(dst_hbm, src_hbm, feat_hbm, out_hbm, cnt_hbm):
        pltpu.sync_copy(zinit, rA)

        def zero_body(i, _):
            off = pl.multiple_of(base + i * 8, 8)
            pltpu.sync_copy(rA.at[pl.ds(0, 8)], acc.at[pl.ds(off, 8)])
            return ()

        lax.fori_loop(0, rows_per_tile // 8, zero_body, (), unroll=False)
        pltpu.sync_copy(zeros1d, cnt1d)
        plsc.subcore_barrier()

        for b, (db, sb, rb, gsem, ssem) in enumerate(bufs):
            pltpu.sync_copy(dst_hbm.at[s, b], db)
            pltpu.sync_copy(src_hbm.at[s, b], sb)
            pltpu.async_copy(feat_hbm.at[sb], rb, gsem)

        def body(jj, _):
            for b, (db, sb, rb, gsem, ssem) in enumerate(bufs):
                j = jj * nbuf + b

                @pl.when(j < NCHUNK)
                def _():
                    pltpu.make_async_copy(feat_hbm.at[sb], rb, gsem).wait()
                    pltpu.async_copy(rb, acc.at[db], ssem, add=True)
                    for k in range(CH // 16):
                        d16 = db[pl.ds(k * 16, 16)]
                        cnts, last = plsc.scan_count(d16)
                        plsc.addupdate_scatter(
                            cnt1d, [d16], cnts.astype(jnp.float32), mask=last)
                    jn = j + nbuf

                    @pl.when(jn < NCHUNK)
                    def _():
                        pltpu.make_async_copy(rb, acc.at[db], ssem).wait()
                        pltpu.sync_copy(dst_hbm.at[s, jn], db)
                        pltpu.sync_copy(src_hbm.at[s, jn], sb)
                        pltpu.async_copy(feat_hbm.at[sb], rb, gsem)
            return ()

        lax.fori_loop(0, (NCHUNK + nbuf - 1) // nbuf, body, (), unroll=False)
        for b, (db, sb, rb, gsem, ssem) in enumerate(bufs):
            pltpu.make_async_copy(rb, acc.at[db], ssem).wait()

        cbase = (c * NS + s) * NPAD

        def pub_body(p, _):
            off = pl.multiple_of(p * 1264, 8)
            pltpu.sync_copy(cnt1d.at[pl.ds(off, 1264)],
                            cnt_stage.at[pl.ds(cbase + off, 1264)])
            return ()

        lax.fori_loop(0, NPAD // 1264, pub_body, (), unroll=False)
        plsc.subcore_barrier()

        def flush_body(i, _):
            off = pl.multiple_of(base + i * 8, 8)
            pltpu.sync_copy(acc.at[pl.ds(off, 8)],
                            out_hbm.at[pl.ds(off, 8)])
            return ()

        lax.fori_loop(0, rows_per_tile // 8, flush_body, (), unroll=False)

        pltpu.sync_copy(
            cnt_stage.at[pl.ds(c * NS * NPAD + s * rows_per_tile,
                               rows_per_tile)], racc.at[pl.ds(0, rows_per_tile)])

        def red_body(t, _):
            off = pl.multiple_of(
                c * NS * NPAD + t * NPAD + s * rows_per_tile, 8)
            pltpu.sync_copy(cnt_stage.at[pl.ds(off, rows_per_tile)],
                            rtmp.at[pl.ds(0, rows_per_tile)])
            for v in range(640 // 16):
                sl = pl.ds(v * 16, 16)
                racc[sl] = racc[sl] + rtmp[sl]
            return ()

        lax.fori_loop(1, NS, red_body, (), unroll=False)
        pltpu.sync_copy(racc.at[pl.ds(0, rows_per_tile)],
                        cnt_hbm.at[pl.ds(s * rows_per_tile, rows_per_tile)])

    @pl.when(c == 0)
    def _():
        run_task(d0, s0, item_f, a0, c0)
        run_task(d1, s1, fitem_f, a1, c1)

    @pl.when(c == 1)
    def _():
        run_task(d2, s2, user_f, a2, c2)
        run_task(d3, s3, fuser_f, a3, c3)


def _segsum_call(item_f, fitem_f, user_f, fuser_f,
                 d0, s0, d1, s1, d2, s2, d3, s3, zinit, zeros1d):
    outa = jax.ShapeDtypeStruct((NPAD, D), jnp.float32)
    outc = jax.ShapeDtypeStruct((NPAD,), jnp.float32)
    rpt = NPAD // NS
    f = pl.kernel(
        _segsum_body,
        out_type=(outa, outa, outa, outa, outc, outc, outc, outc),
        mesh=_sc_mesh(),
        compiler_params=pltpu.CompilerParams(needs_layout_passes=False),
        scratch_types=(
            [pltpu.VMEM_SHARED((NPAD, D), jnp.float32),
             pltpu.HBM((NC * NS * NPAD,), jnp.float32)]
            + [pltpu.VMEM((CH,), jnp.int32)] * 4
            + [pltpu.VMEM((CH, D), jnp.float32)] * 2
            + [pltpu.VMEM((NPAD,), jnp.float32),
               pltpu.VMEM((640,), jnp.float32),
               pltpu.VMEM((640,), jnp.float32)]
            + [pltpu.SemaphoreType.DMA] * 4
        ),
    )
    return f(item_f, fitem_f, user_f, fuser_f,
             d0, s0, d1, s1, d2, s2, d3, s3, zinit, zeros1d)


def _gather_body(a0, a1, a2, a3, cn0, cn1, cn2, cn3, uidx_hbm, iidx_hbm,
                 g0, g1, g2, g3, o0, o1, o2, o3,
                 idxu, idxi, rows, cntv, cbuf, sem):
    c = lax.axis_index("c")
    s = lax.axis_index("s")
    w = c * NS + s
    pltpu.sync_copy(uidx_hbm.at[w], idxu)
    pltpu.sync_copy(iidx_hbm.at[w], idxi)

    def one_table(acc_hbm, cnt_hbm, idx_ref, out_hbm, co_hbm):
        pltpu.async_copy(acc_hbm.at[idx_ref], rows, sem).wait()
        pltpu.sync_copy(rows, out_hbm.at[pl.ds(w * BPW, BPW)])
        pltpu.sync_copy(cnt_hbm, cntv)
        for g in range(BPW // 16):
            iv = idx_ref[pl.ds(g * 16, 16)]
            cv = plsc.load_gather(cntv, [iv])
            cbuf[pl.ds(g * 16, 16)] = cv
        pltpu.sync_copy(cbuf, co_hbm.at[w])

    one_table(a0, cn0, idxu, g0, o0)
    one_table(a1, cn1, idxu, g1, o1)
    one_table(a2, cn2, idxi, g2, o2)
    one_table(a3, cn3, idxi, g3, o3)


def _gather_call(a0, a1, a2, a3, cn0, cn1, cn2, cn3, uidx, iidx):
    outg = jax.ShapeDtypeStruct((B, D), jnp.float32)
    outc = jax.ShapeDtypeStruct((NW, BPW), jnp.float32)
    f = pl.kernel(
        _gather_body,
        out_type=(outg, outg, outg, outg, outc, outc, outc, outc),
        mesh=_sc_mesh(),
        compiler_params=pltpu.CompilerParams(needs_layout_passes=False),
        scratch_types=[
            pltpu.VMEM((BPW,), jnp.int32),
            pltpu.VMEM((BPW,), jnp.int32),
            pltpu.VMEM((BPW, D), jnp.float32),
            pltpu.VMEM((NPAD,), jnp.float32),
            pltpu.VMEM((BPW,), jnp.float32),
            pltpu.SemaphoreType.DMA,
        ],
    )
    return f(a0, a1, a2, a3, cn0, cn1, cn2, cn3, uidx, iidx)


def _dense_body(u_ref, i_ref, g0, g1, g2, g3, c0, c1, c2, c3,
                wgut, bgu, wgvt, bgv, wlint, blin, wsubt, bsub, wdt, bd,
                prob_ref, label_ref):
    f32 = jnp.float32
    su = jnp.mean(u_ref[...], axis=0, keepdims=True)
    si = jnp.mean(i_ref[...], axis=0, keepdims=True)
    scat = jnp.concatenate([su, si], axis=1)
    s_two = jax.nn.sigmoid(
        jnp.dot(scat, wlint[...], preferred_element_type=f32) + blin[...])
    t = jnp.dot(s_two, wdt[...], preferred_element_type=f32)

    def gat(g, cn, wt, b):
        m = g[...] / jnp.maximum(cn[...], 1.0)
        return jax.nn.relu(jnp.dot(m, wt[...], preferred_element_type=f32) + b[...])

    ru = gat(g0, c0, wgut, bgu)
    fu = gat(g1, c1, wgut, bgu)
    ri = gat(g2, c2, wgvt, bgv)
    fi = gat(g3, c3, wgvt, bgv)

    wsu = wsubt[0:D, :]
    wsi = wsubt[D:2 * D, :]
    real_sub = jax.nn.sigmoid(
        jnp.dot(ru, wsu, preferred_element_type=f32)
        + jnp.dot(ri, wsi, preferred_element_type=f32) + bsub[...])
    fake_sub = jax.nn.sigmoid(
        jnp.dot(fu, wsu, preferred_element_type=f32)
        + jnp.dot(fi, wsi, preferred_element_type=f32) + bsub[...])

    b0 = bd[0, 0]
    real_prob = jax.nn.sigmoid(
        jnp.sum(real_sub * t, axis=1, keepdims=True) + b0)
    fake_prob = jax.nn.sigmoid(
        jnp.sum(fake_sub * t, axis=1, keepdims=True) + b0)
    prob_ref[pl.ds(0, B), :] = real_prob
    prob_ref[pl.ds(B, B), :] = fake_prob
    label_ref[pl.ds(0, B), :] = jnp.ones((B, 1), f32)
    label_ref[pl.ds(B, B), :] = jnp.zeros((B, 1), f32)


def _dense_call(uh, ih, g0, g1, g2, g3, c0, c1, c2, c3,
                wgut, bgu, wgvt, bgv, wlint, blin, wsubt, bsub, wdt, bd):
    return pl.pallas_call(
        _dense_body,
        out_shape=(
            jax.ShapeDtypeStruct((2 * B, 1), jnp.float32),
            jax.ShapeDtypeStruct((2 * B, 1), jnp.float32),
        ),
    )(uh, ih, g0, g1, g2, g3, c0, c1, c2, c3,
      wgut, bgu, wgvt, bgv, wlint, blin, wsubt, bsub, wdt, bd)


def kernel(user_hidden_out, item_hidden_out, fake_user_hidden_out,
           fake_item_hidden_out, UV_adj, VU_adj, CUV_adj, CVU_adj,
           user_One, item_One, Wgu, bgu, Wgv, bgv, W_lin, b_lin,
           W_sub, b_sub, W_disc, b_disc):
    f32 = jnp.float32

    def prep(adj):
        pad = EPAD - E
        dst = jnp.concatenate(
            [adj[0].astype(jnp.int32), jnp.full((pad,), N, jnp.int32)])
        src = jnp.concatenate(
            [adj[1].astype(jnp.int32), jnp.zeros((pad,), jnp.int32)])
        return dst.reshape(NS, NCHUNK, CH), src.reshape(NS, NCHUNK, CH)

    d0, s0 = prep(UV_adj)
    d1, s1 = prep(CUV_adj)
    d2, s2 = prep(VU_adj)
    d3, s3 = prep(CVU_adj)
    zinit = jnp.zeros((128, 128), f32)
    zeros1d = jnp.zeros((NPAD,), f32)

    a0, a1, a2, a3, c0, c1, c2, c3 = _segsum_call(
        item_hidden_out, fake_item_hidden_out,
        user_hidden_out, fake_user_hidden_out,
        d0, s0, d1, s1, d2, s2, d3, s3, zinit, zeros1d)

    uidx = user_One.astype(jnp.int32).reshape(NW, BPW)
    iidx = item_One.astype(jnp.int32).reshape(NW, BPW)
    g0, g1, g2, g3, o0, o1, o2, o3 = _gather_call(
        a0, a1, a2, a3, c0, c1, c2, c3, uidx, iidx)

    prob2, label2 = _dense_call(
        user_hidden_out, item_hidden_out, g0, g1, g2, g3,
        o0.reshape(B, 1), o1.reshape(B, 1), o2.reshape(B, 1), o3.reshape(B, 1),
        Wgu.T, bgu.reshape(1, D), Wgv.T, bgv.reshape(1, D),
        W_lin.T, b_lin.reshape(1, D), W_sub.T, b_sub.reshape(1, D),
        W_disc[0].T, b_disc.reshape(1, 1))

    return prob2.reshape(2 * B), label2.reshape(2 * B)
